# baseline (speedup 1.0000x reference)
import functools

import jax
import jax.numpy as jnp
from jax import lax
from jax.experimental import pallas as pl
from jax.experimental.pallas import tpu as pltpu
from jax.experimental.pallas import tpu_sc as plsc

_W = 128


@functools.partial(jax.jit, static_argnums=(2, 3))
def _sc_gather(table, flat_idx, n, d):
    mesh = plsc.VectorSubcoreMesh(core_axis_name="c", subcore_axis_name="s")

    @functools.partial(
        pl.kernel,
        out_type=jax.ShapeDtypeStruct((n, d), table.dtype),
        mesh=mesh,
    )
    def gather_kernel(table_hbm, idx_hbm, out_hbm):
        def body(i_vmem, o_vmem):
            pltpu.sync_copy(table_hbm.at[i_vmem.at[0]], o_vmem)

        pltpu.emit_pipeline(
            body,
            grid=(n // _W,),
            in_specs=[pl.BlockSpec((1, _W), lambda i: (0, i))],
            out_specs=[pl.BlockSpec((_W, d), lambda i: (i, 0))],
            core_axis_name=("c", "s"),
            dimension_semantics=(pltpu.PARALLEL,),
        )(idx_hbm, out_hbm)

    return gather_kernel(table, flat_idx)


def kernel(input_batch, seq_lengths, targets_batch, table):
    B, L = input_batch.shape
    V, D = table.shape

    lengths = jnp.maximum(seq_lengths, 1)
    perm = jnp.argsort(-lengths)
    sorted_lengths = lengths[perm]

    pos = jnp.arange(L, dtype=jnp.int32)[None, :]
    tokens = jnp.where(
        pos < sorted_lengths[:, None],
        input_batch[perm].astype(jnp.int32),
        0,
    )
    flat_idx = tokens.reshape(1, B * L)

    embedded = _sc_gather(table, flat_idx, B * L, D).reshape(B, L, D)
    return embedded, sorted_lengths.astype(jnp.float32), targets_batch[perm]
